# baseline (speedup 1.0000x reference)
import functools

import jax
import jax.numpy as jnp
from jax import lax
from jax.experimental import pallas as pl
from jax.experimental.pallas import tpu as pltpu
from jax.experimental.pallas import tpu_sc as plsc

EVENT_DIM = 32
WORD_DIM = 64

NC = 2
NS = 16
NW = NC * NS

BLK = 128
NBUF = 5
LOOKAHEAD = 3


def _table_loop(tab_hbm, idx_v, rows_v, out_hbm, base, num_blocks, gsem, wsem):
    for g in range(LOOKAHEAD):
        pltpu.async_copy(tab_hbm.at[idx_v.at[g]], rows_v.at[g], gsem.at[g])

    def one_iter(g, j, do_drain, do_fire):
        pltpu.make_async_copy(
            tab_hbm.at[idx_v.at[g]], rows_v.at[j], gsem.at[j]
        ).wait()
        pltpu.async_copy(
            rows_v.at[j], out_hbm.at[pl.ds(base + g * BLK, BLK)], wsem.at[j]
        )
        gf = g + LOOKAHEAD
        bf = (j + LOOKAHEAD) % NBUF
        if do_drain:
            pltpu.make_async_copy(
                rows_v.at[bf],
                out_hbm.at[pl.ds(base + (gf - NBUF) * BLK, BLK)],
                wsem.at[bf],
            ).wait()
        if do_fire:
            pltpu.async_copy(
                tab_hbm.at[idx_v.at[gf]], rows_v.at[bf], gsem.at[bf]
            )

    num_steps = num_blocks // NBUF
    for j in range(NBUF):
        g = j
        one_iter(g, j, do_drain=g + LOOKAHEAD >= NBUF, do_fire=True)

    def step(s, _):
        for j in range(NBUF):
            one_iter(s * NBUF + j, j, do_drain=True, do_fire=True)
        return _
    lax.fori_loop(1, num_steps - 1, step, None)

    for j in range(NBUF):
        g = (num_steps - 1) * NBUF + j
        gf = g + LOOKAHEAD
        one_iter(g, j, do_drain=gf < num_blocks, do_fire=gf < num_blocks)
    for j in range(NBUF):
        g_last = num_blocks - NBUF + j
        pltpu.make_async_copy(
            rows_v.at[j], out_hbm.at[pl.ds(base + g_last * BLK, BLK)], wsem.at[j]
        ).wait()


def _emb_kernel(total, k_per_w):
    mesh = plsc.VectorSubcoreMesh(core_axis_name="c", subcore_axis_name="s")

    @functools.partial(
        pl.kernel,
        out_type=(
            jax.ShapeDtypeStruct((total, EVENT_DIM), jnp.float32),
            jax.ShapeDtypeStruct((total, WORD_DIM), jnp.float32),
        ),
        mesh=mesh,
        compiler_params=pltpu.CompilerParams(use_tc_tiling_on_sc=False),
        scratch_types=[
            pltpu.VMEM((k_per_w, BLK), jnp.int32),
            pltpu.VMEM((k_per_w, BLK), jnp.int32),
            pltpu.VMEM((NBUF, BLK, EVENT_DIM), jnp.float32),
            pltpu.VMEM((NBUF, BLK, WORD_DIM), jnp.float32),
            pltpu.SemaphoreType.DMA((NBUF,)),
            pltpu.SemaphoreType.DMA((NBUF,)),
        ],
    )
    def k(ev_idx_hbm, wd_idx_hbm, ev_tab, wd_tab, ev_out, wd_out,
          ev_idx_v, wd_idx_v, ev_rows, wd_rows, gsem, wsem):
        wid = lax.axis_index("s") * NC + lax.axis_index("c")
        base = wid * (k_per_w * BLK)
        pltpu.sync_copy(ev_idx_hbm.at[wid], ev_idx_v)
        pltpu.sync_copy(wd_idx_hbm.at[wid], wd_idx_v)
        _table_loop(ev_tab, ev_idx_v, ev_rows, ev_out, base, k_per_w, gsem, wsem)
        _table_loop(wd_tab, wd_idx_v, wd_rows, wd_out, base, k_per_w, gsem, wsem)

    return k


def kernel(event_ids, word_ids, event_table, word_table):
    batch, hist = event_ids.shape
    total = batch * hist
    k_per_w = total // (NW * BLK)
    ev_idx = event_ids.reshape(NW, k_per_w, BLK).astype(jnp.int32) & 0x7FFFFFFF
    wd_idx = word_ids.reshape(NW, k_per_w, BLK).astype(jnp.int32) & 0x7FFFFFFF
    one = jnp.float32(1.0 + 1e-7)
    ev_out, wd_out = _emb_kernel(total, k_per_w)(
        ev_idx, wd_idx, event_table * one, word_table * one)
    return (
        ev_out.reshape(batch, hist, EVENT_DIM) * one,
        wd_out.reshape(batch, hist, WORD_DIM) * one,
    )


# speedup vs baseline: 2.2193x; 2.2193x over previous
import functools

import jax
import jax.numpy as jnp
from jax import lax
from jax.experimental import pallas as pl
from jax.experimental.pallas import tpu as pltpu
from jax.experimental.pallas import tpu_sc as plsc

EVENT_DIM = 32
WORD_DIM = 64

NC = 2
NS = 16
NW = NC * NS

BLK = 128
NBUF = 5
LOOKAHEAD = 3


def _table_loop(tab_hbm, idx_v, rows_v, out_hbm, base, num_blocks, gsem, wsem):
    for g in range(LOOKAHEAD):
        pltpu.async_copy(
            tab_hbm.at[idx_v.at[pl.ds(g * BLK, BLK)]], rows_v.at[g], gsem.at[g]
        )

    def one_iter(g, j, do_drain, do_fire):
        pltpu.make_async_copy(
            tab_hbm.at[idx_v.at[pl.ds(g * BLK, BLK)]], rows_v.at[j], gsem.at[j]
        ).wait()
        pltpu.async_copy(
            rows_v.at[j], out_hbm.at[pl.ds(base + g * BLK, BLK)], wsem.at[j]
        )
        gf = g + LOOKAHEAD
        bf = (j + LOOKAHEAD) % NBUF
        if do_drain:
            pltpu.make_async_copy(
                rows_v.at[bf],
                out_hbm.at[pl.ds(base + (gf - NBUF) * BLK, BLK)],
                wsem.at[bf],
            ).wait()
        if do_fire:
            pltpu.async_copy(
                tab_hbm.at[idx_v.at[pl.ds(gf * BLK, BLK)]], rows_v.at[bf],
                gsem.at[bf],
            )

    num_steps = num_blocks // NBUF
    for j in range(NBUF):
        g = j
        one_iter(g, j, do_drain=g + LOOKAHEAD >= NBUF, do_fire=True)

    def step(s, _):
        for j in range(NBUF):
            one_iter(s * NBUF + j, j, do_drain=True, do_fire=True)
        return _
    lax.fori_loop(1, num_steps - 1, step, None)

    for j in range(NBUF):
        g = (num_steps - 1) * NBUF + j
        gf = g + LOOKAHEAD
        one_iter(g, j, do_drain=gf < num_blocks, do_fire=gf < num_blocks)
    for j in range(NBUF):
        g_last = num_blocks - NBUF + j
        pltpu.make_async_copy(
            rows_v.at[j], out_hbm.at[pl.ds(base + g_last * BLK, BLK)], wsem.at[j]
        ).wait()


def _emb_kernel(total, k_per_w):
    mesh = plsc.VectorSubcoreMesh(core_axis_name="c", subcore_axis_name="s")

    @functools.partial(
        pl.kernel,
        out_type=(
            jax.ShapeDtypeStruct((total, EVENT_DIM), jnp.float32),
            jax.ShapeDtypeStruct((total, WORD_DIM), jnp.float32),
        ),
        mesh=mesh,
        compiler_params=pltpu.CompilerParams(use_tc_tiling_on_sc=False),
        scratch_types=[
            pltpu.VMEM((k_per_w * BLK,), jnp.int32),
            pltpu.VMEM((k_per_w * BLK,), jnp.int32),
            pltpu.VMEM((NBUF, BLK, EVENT_DIM), jnp.float32),
            pltpu.VMEM((NBUF, BLK, WORD_DIM), jnp.float32),
            pltpu.SemaphoreType.DMA((NBUF,)),
            pltpu.SemaphoreType.DMA((NBUF,)),
        ],
    )
    def k(ev_idx_hbm, wd_idx_hbm, ev_tab, wd_tab, ev_out, wd_out,
          ev_idx_v, wd_idx_v, ev_rows, wd_rows, gsem, wsem):
        wid = lax.axis_index("s") * NC + lax.axis_index("c")
        base = wid * (k_per_w * BLK)
        pltpu.sync_copy(ev_idx_hbm.at[pl.ds(base, k_per_w * BLK)], ev_idx_v)
        pltpu.sync_copy(wd_idx_hbm.at[pl.ds(base, k_per_w * BLK)], wd_idx_v)
        _table_loop(ev_tab, ev_idx_v, ev_rows, ev_out, base, k_per_w, gsem, wsem)
        _table_loop(wd_tab, wd_idx_v, wd_rows, wd_out, base, k_per_w, gsem, wsem)

    return k


def kernel(event_ids, word_ids, event_table, word_table):
    batch, hist = event_ids.shape
    total = batch * hist
    k_per_w = total // (NW * BLK)
    ev_idx = event_ids.T.reshape(total).astype(jnp.int32)
    wd_idx = word_ids.T.reshape(total).astype(jnp.int32)
    ev_out, wd_out = _emb_kernel(total, k_per_w)(
        ev_idx, wd_idx, event_table, word_table)
    return (
        jnp.swapaxes(ev_out.reshape(hist, batch, EVENT_DIM), 0, 1),
        jnp.swapaxes(wd_out.reshape(hist, batch, WORD_DIM), 0, 1),
    )


# speedup vs baseline: 2.3374x; 1.0532x over previous
import functools

import jax
import jax.numpy as jnp
from jax import lax
from jax.experimental import pallas as pl
from jax.experimental.pallas import tpu as pltpu
from jax.experimental.pallas import tpu_sc as plsc

EVENT_DIM = 32
WORD_DIM = 64

NC = 2
NS = 16
NW = NC * NS

BLK = 128
NBUF = 5
LOOKAHEAD = 3


def _table_loop(tab_hbm, idx_v, rows_v, out_hbm, base, num_blocks, gsem, wsem):
    for g in range(LOOKAHEAD):
        pltpu.async_copy(
            tab_hbm.at[idx_v.at[pl.ds(g * BLK, BLK)]], rows_v.at[g], gsem.at[g]
        )

    def one_iter(g, j, do_drain, do_fire):
        pltpu.make_async_copy(
            tab_hbm.at[idx_v.at[pl.ds(g * BLK, BLK)]], rows_v.at[j], gsem.at[j]
        ).wait()
        pltpu.async_copy(
            rows_v.at[j], out_hbm.at[pl.ds(base + g * BLK, BLK)], wsem.at[j]
        )
        gf = g + LOOKAHEAD
        bf = (j + LOOKAHEAD) % NBUF
        if do_drain:
            pltpu.make_async_copy(
                rows_v.at[bf],
                out_hbm.at[pl.ds(base + (gf - NBUF) * BLK, BLK)],
                wsem.at[bf],
            ).wait()
        if do_fire:
            pltpu.async_copy(
                tab_hbm.at[idx_v.at[pl.ds(gf * BLK, BLK)]], rows_v.at[bf],
                gsem.at[bf],
            )

    num_steps = num_blocks // NBUF
    for j in range(NBUF):
        g = j
        one_iter(g, j, do_drain=g + LOOKAHEAD >= NBUF, do_fire=True)

    def step(s, _):
        for j in range(NBUF):
            one_iter(s * NBUF + j, j, do_drain=True, do_fire=True)
        return _
    lax.fori_loop(1, num_steps - 1, step, None)

    for j in range(NBUF):
        g = (num_steps - 1) * NBUF + j
        gf = g + LOOKAHEAD
        one_iter(g, j, do_drain=gf < num_blocks, do_fire=gf < num_blocks)
    for j in range(NBUF):
        g_last = num_blocks - NBUF + j
        pltpu.make_async_copy(
            rows_v.at[j], out_hbm.at[pl.ds(base + g_last * BLK, BLK)], wsem.at[j]
        ).wait()


def _emb_kernel(total, k_per_w, dim):
    mesh = plsc.VectorSubcoreMesh(core_axis_name="c", subcore_axis_name="s")

    @functools.partial(
        pl.kernel,
        out_type=jax.ShapeDtypeStruct((total, dim), jnp.float32),
        mesh=mesh,
        compiler_params=pltpu.CompilerParams(use_tc_tiling_on_sc=False),
        scratch_types=[
            pltpu.VMEM((k_per_w * BLK,), jnp.int32),
            pltpu.VMEM((NBUF, BLK, dim), jnp.float32),
            pltpu.SemaphoreType.DMA((NBUF,)),
            pltpu.SemaphoreType.DMA((NBUF,)),
        ],
    )
    def k(idx_hbm, tab, out, idx_v, rows, gsem, wsem):
        wid = lax.axis_index("s") * NC + lax.axis_index("c")
        base = wid * (k_per_w * BLK)
        pltpu.sync_copy(idx_hbm.at[pl.ds(base, k_per_w * BLK)], idx_v)
        _table_loop(tab, idx_v, rows, out, base, k_per_w, gsem, wsem)

    return k


def kernel(event_ids, word_ids, event_table, word_table):
    batch, hist = event_ids.shape
    total = batch * hist
    k_per_w = total // (NW * BLK)
    ev_idx = event_ids.T.reshape(total).astype(jnp.int32)
    wd_idx = word_ids.T.reshape(total).astype(jnp.int32)
    ev_out = _emb_kernel(total, k_per_w, EVENT_DIM)(ev_idx, event_table)
    wd_out = _emb_kernel(total, k_per_w, WORD_DIM)(wd_idx, word_table)
    return (
        jnp.swapaxes(ev_out.reshape(hist, batch, EVENT_DIM), 0, 1),
        jnp.swapaxes(wd_out.reshape(hist, batch, WORD_DIM), 0, 1),
    )
